# baseline (speedup 1.0000x reference)
import functools

import jax
import jax.numpy as jnp
import numpy as np
from jax import lax
from jax.experimental import pallas as pl
from jax.experimental.pallas import tpu as pltpu
from jax.experimental.pallas import tpu_sc as plsc

N_NODES = 10000
N_EDGES = 320000
D = 128
DH = D // 2

NC = 2
NS = 16
NPAD = 10080
ROWS_PER_TILE = NPAD // NS
EDGES_PER_TILE = N_EDGES // NS
K = 80
CHUNKS = EDGES_PER_TILE // K

_MESH = plsc.VectorSubcoreMesh(
    core_axis_name="c", subcore_axis_name="s", num_cores=NC, num_subcores=NS
)


_ZR = 126

_GDN = lax.GatherDimensionNumbers(
    offset_dims=(), collapsed_slice_dims=(0,), start_index_map=(0,)
)


def _make_spmm(table_rows: int):
    @functools.partial(
        pl.kernel,
        out_type=jax.ShapeDtypeStruct((NPAD, D), jnp.float32),
        mesh=_MESH,
        compiler_params=pltpu.CompilerParams(
            use_tc_tiling_on_sc=False, needs_layout_passes=False
        ),
        scratch_types=[
            pltpu.VMEM((CHUNKS, K), jnp.int32),
            pltpu.VMEM((CHUNKS, K), jnp.int32),
            pltpu.VMEM((CHUNKS, K), jnp.float32),
            pltpu.VMEM((4, K, DH), jnp.bfloat16),
            pltpu.VMEM((2, K, DH), jnp.float32),
            pltpu.VMEM((_ZR, DH), jnp.float32),
            pltpu.VMEM_SHARED((NPAD, DH), jnp.float32),
            pltpu.SemaphoreType.DMA,
            pltpu.SemaphoreType.DMA,
            pltpu.SemaphoreType.DMA,
        ],
    )
    def spmm(
        table, src, dst, w, out, src_v, dst_v, w_v, rows_v, rowsf_v, zero_v,
        acc, gsem, ssem0, ssem1,
    ):
        ssems = (ssem0, ssem1)
        c = lax.axis_index("c")
        s = lax.axis_index("s")

        row0 = s * CHUNKS
        pltpu.sync_copy(src.at[pl.ds(row0, CHUNKS)], src_v)
        pltpu.sync_copy(dst.at[pl.ds(row0, CHUNKS)], dst_v)
        pltpu.sync_copy(w.at[pl.ds(row0, CHUNKS)], w_v)

        def gather_start(i, b):
            pltpu.async_copy(table.at[c].at[src_v.at[i]], rows_v.at[b], gsem)

        def gather_wait(b):
            pltpu.make_async_copy(
                table.at[c].at[src_v.at[0]], rows_v.at[b], gsem
            ).wait()

        def scatter_start(i, sb):
            pltpu.async_copy(
                rowsf_v.at[sb], acc.at[dst_v.at[i]], ssems[sb], add=True
            )

        def scatter_wait(sb):
            pltpu.make_async_copy(
                rowsf_v.at[sb], acc.at[dst_v.at[0]], ssems[sb]
            ).wait()

        def scale(i, b, sb):
            for g in range(K // 16):
                w16 = w_v[i, pl.ds(g * 16, 16)]
                for t in range(16):
                    bidx = jnp.full((16, 1), t, dtype=jnp.int32)
                    wv = lax.gather(
                        w16, bidx, _GDN, slice_sizes=(1,),
                        mode=lax.GatherScatterMode.PROMISE_IN_BOUNDS,
                    )
                    e = g * 16 + t
                    for j in range(DH // 32):
                        v32 = rows_v[b, e, pl.ds(j * 32, 32)]
                        lo, hi = plsc.unpack(
                            v32, format=plsc.PackFormat.INTERLEAVED
                        )
                        rowsf_v[sb, e, pl.ds(j * 32, 16)] = lo * wv
                        rowsf_v[sb, e, pl.ds(j * 32 + 16, 16)] = hi * wv

        gather_start(0, 0)
        gather_start(1, 1)
        gather_start(2, 2)

        zeros16 = jnp.zeros((16,), jnp.float32)

        def zero_body(i, carry):
            for j in range(DH // 16):
                zero_v[i, pl.ds(j * 16, 16)] = zeros16
            return carry

        lax.fori_loop(0, _ZR, zero_body, 0)
        for r in range(ROWS_PER_TILE // _ZR):
            pltpu.sync_copy(
                zero_v, acc.at[pl.ds(s * ROWS_PER_TILE + r * _ZR, _ZR)]
            )
        plsc.subcore_barrier()

        def make_slot_body(guarded):
            def slot_body(t, carry):
                for b in range(4):
                    i = 4 * t + b
                    pb = (b + 3) % 4
                    sb = b % 2

                    def do_wait():
                        scatter_wait(sb)

                    def do_scale():
                        gather_wait(b)
                        scale(i, b, sb)

                    def do_gather():
                        gather_start(i + 3, pb)

                    def do_scatter():
                        scatter_start(i, sb)

                    if guarded:
                        pl.when(jnp.logical_and(i >= 2, i <= CHUNKS + 1))(
                            do_wait
                        )
                        pl.when(i < CHUNKS)(do_scale)
                        pl.when(i + 3 < CHUNKS)(do_gather)
                        pl.when(i < CHUNKS)(do_scatter)
                    else:
                        do_wait()
                        do_scale()
                        do_gather()
                        do_scatter()
                return carry

            return slot_body

        lax.fori_loop(0, 1, make_slot_body(True), 0)
        lax.fori_loop(1, 61, make_slot_body(False), 0)
        lax.fori_loop(61, (CHUNKS + 5) // 4, make_slot_body(True), 0)
        plsc.subcore_barrier()

        pltpu.sync_copy(
            acc.at[pl.ds(s * ROWS_PER_TILE, ROWS_PER_TILE)],
            out.at[pl.ds(s * ROWS_PER_TILE, ROWS_PER_TILE), pl.ds(c * DH, DH)],
        )

    return spmm


_spmm_l1 = _make_spmm(N_NODES)
_spmm_l2 = _make_spmm(NPAD)


def _mm1_body(x_ref, w_ref, b_ref, o_ref):
    o_ref[0] = (
        jnp.dot(x_ref[...], w_ref[0], preferred_element_type=jnp.float32)
        + b_ref[0]
    ).astype(jnp.bfloat16)


def _mm2_body(p_ref, w_ref, b_ref, o_ref):
    h = jnp.maximum(p_ref[...], 0.0)
    o_ref[0] = (
        jnp.dot(h, w_ref[0], preferred_element_type=jnp.float32) + b_ref[0]
    ).astype(jnp.bfloat16)


_PERM = np.array(
    [32 * g + 16 * h + p for g in range(2) for p in range(16) for h in range(2)],
    dtype=np.int32,
)


def _split_wb(W, b):
    Ws = jnp.stack([W[:, 0:DH], W[:, DH:D]])[:, :, _PERM]
    bs = b.reshape(NC, 1, DH)[:, :, _PERM]
    return Ws, bs


def _mm1(x, W, b):
    Ws, bs = _split_wb(W, b)
    return pl.pallas_call(
        _mm1_body,
        grid=(NC,),
        in_specs=[
            pl.BlockSpec((N_NODES, D), lambda c: (0, 0)),
            pl.BlockSpec((1, D, DH), lambda c: (c, 0, 0)),
            pl.BlockSpec((1, 1, DH), lambda c: (c, 0, 0)),
        ],
        out_specs=pl.BlockSpec((1, N_NODES, DH), lambda c: (c, 0, 0)),
        out_shape=jax.ShapeDtypeStruct((NC, N_NODES, DH), jnp.bfloat16),
    )(x, Ws, bs)


def _mm2(p, W, b):
    Ws, bs = _split_wb(W, b)
    return pl.pallas_call(
        _mm2_body,
        grid=(NC,),
        in_specs=[
            pl.BlockSpec((NPAD, D), lambda c: (0, 0)),
            pl.BlockSpec((1, D, DH), lambda c: (c, 0, 0)),
            pl.BlockSpec((1, 1, DH), lambda c: (c, 0, 0)),
        ],
        out_specs=pl.BlockSpec((1, NPAD, DH), lambda c: (c, 0, 0)),
        out_shape=jax.ShapeDtypeStruct((NC, NPAD, DH), jnp.bfloat16),
    )(p, Ws, bs)


@jax.jit
def kernel(x, edge_index, edge_weight, W1, b1, W2, b2):
    dst = edge_index[0].reshape(N_EDGES // K, K)
    src = edge_index[1].reshape(N_EDGES // K, K)
    w2d = edge_weight.reshape(N_EDGES // K, K)

    hw1 = _mm1(x, W1, b1)
    p = _spmm_l1(hw1, src, dst, w2d)
    hw2 = _mm2(p, W2, b2)
    q = _spmm_l2(hw2, src, dst, w2d)
    return q[:N_NODES]


# speedup vs baseline: 1.3614x; 1.3614x over previous
import functools

import jax
import jax.numpy as jnp
import numpy as np
from jax import lax
from jax.experimental import pallas as pl
from jax.experimental.pallas import tpu as pltpu
from jax.experimental.pallas import tpu_sc as plsc

N_NODES = 10000
N_EDGES = 320000
D = 128
DH = D // 2

NC = 2
NS = 16
NPAD = 10080
ROWS_PER_TILE = NPAD // NS
EDGES_PER_TILE = N_EDGES // NS
K = 80
CHUNKS = EDGES_PER_TILE // K

_MESH = plsc.VectorSubcoreMesh(
    core_axis_name="c", subcore_axis_name="s", num_cores=NC, num_subcores=NS
)


_ZR = 126

_GDN = lax.GatherDimensionNumbers(
    offset_dims=(), collapsed_slice_dims=(0,), start_index_map=(0,)
)


def _make_spmm(table_rows: int):
    @functools.partial(
        pl.kernel,
        out_type=jax.ShapeDtypeStruct((NPAD, D), jnp.float32),
        mesh=_MESH,
        compiler_params=pltpu.CompilerParams(
            use_tc_tiling_on_sc=False, needs_layout_passes=False
        ),
        scratch_types=[
            pltpu.VMEM((CHUNKS, K), jnp.int32),
            pltpu.VMEM((CHUNKS, K), jnp.int32),
            pltpu.VMEM((CHUNKS, K), jnp.float32),
            pltpu.VMEM((4, K, DH), jnp.bfloat16),
            pltpu.VMEM((2, K, DH), jnp.float32),
            pltpu.VMEM((_ZR, DH), jnp.float32),
            pltpu.VMEM_SHARED((NPAD, DH), jnp.float32),
            pltpu.SemaphoreType.DMA,
            pltpu.SemaphoreType.DMA,
            pltpu.SemaphoreType.DMA,
        ],
    )
    def spmm(
        table, src, dst, w, out, src_v, dst_v, w_v, rows_v, rowsf_v, zero_v,
        acc, gsem, ssem0, ssem1,
    ):
        ssems = (ssem0, ssem1)
        c = lax.axis_index("c")
        s = lax.axis_index("s")

        row0 = s * CHUNKS
        pltpu.sync_copy(src.at[pl.ds(row0, CHUNKS)], src_v)
        pltpu.sync_copy(dst.at[pl.ds(row0, CHUNKS)], dst_v)
        pltpu.sync_copy(w.at[pl.ds(row0, CHUNKS)], w_v)

        def gather_start(i, b):
            pltpu.async_copy(table.at[c].at[src_v.at[i]], rows_v.at[b], gsem)

        def gather_wait(b):
            pltpu.make_async_copy(
                table.at[c].at[src_v.at[0]], rows_v.at[b], gsem
            ).wait()

        def scatter_start(i, sb):
            pltpu.async_copy(
                rowsf_v.at[sb], acc.at[dst_v.at[i]], ssems[sb], add=True
            )

        def scatter_wait(sb):
            pltpu.make_async_copy(
                rowsf_v.at[sb], acc.at[dst_v.at[0]], ssems[sb]
            ).wait()

        def scale(i, b, sb):
            for g in range(K // 16):
                w16 = w_v[i, pl.ds(g * 16, 16)]
                for t in range(16):
                    bidx = jnp.full((16, 1), t, dtype=jnp.int32)
                    wv = lax.gather(
                        w16, bidx, _GDN, slice_sizes=(1,),
                        mode=lax.GatherScatterMode.PROMISE_IN_BOUNDS,
                    )
                    e = g * 16 + t
                    for j in range(DH // 32):
                        v32 = rows_v[b, e, pl.ds(j * 32, 32)]
                        lo, hi = plsc.unpack(
                            v32, format=plsc.PackFormat.INTERLEAVED
                        )
                        rowsf_v[sb, e, pl.ds(j * 32, 16)] = lo * wv
                        rowsf_v[sb, e, pl.ds(j * 32 + 16, 16)] = hi * wv

        gather_start(0, 0)
        gather_start(1, 1)
        gather_start(2, 2)

        zeros16 = jnp.zeros((16,), jnp.float32)

        def zero_body(i, carry):
            for j in range(DH // 16):
                zero_v[i, pl.ds(j * 16, 16)] = zeros16
            return carry

        lax.fori_loop(0, _ZR, zero_body, 0)
        for r in range(ROWS_PER_TILE // _ZR):
            pltpu.sync_copy(
                zero_v, acc.at[pl.ds(s * ROWS_PER_TILE + r * _ZR, _ZR)]
            )
        plsc.subcore_barrier()

        def slot_body(t, carry):
            for b in range(4):
                i = 4 * t + b
                pb = (b + 3) % 4
                sb = b % 2

                @pl.when(jnp.logical_and(i >= 2, i <= CHUNKS + 1))
                def _():
                    scatter_wait(sb)

                @pl.when(i < CHUNKS)
                def _():
                    gather_wait(b)
                    scale(i, b, sb)

                @pl.when(i + 3 < CHUNKS)
                def _():
                    gather_start(i + 3, pb)

                @pl.when(i < CHUNKS)
                def _():
                    scatter_start(i, sb)
            return carry

        lax.fori_loop(0, (CHUNKS + 5) // 4, slot_body, 0)
        plsc.subcore_barrier()

        pltpu.sync_copy(
            acc.at[pl.ds(s * ROWS_PER_TILE, ROWS_PER_TILE)],
            out.at[pl.ds(s * ROWS_PER_TILE, ROWS_PER_TILE), pl.ds(c * DH, DH)],
        )

    return spmm


_spmm_l1 = _make_spmm(N_NODES)
_spmm_l2 = _make_spmm(NPAD)


def _mm1_body(x_ref, w_ref, b_ref, o_ref):
    o_ref[0] = (
        jnp.dot(x_ref[...], w_ref[0], preferred_element_type=jnp.float32)
        + b_ref[0]
    ).astype(jnp.bfloat16)


def _mm2_body(p_ref, w_ref, b_ref, o_ref):
    h = jnp.maximum(p_ref[...], 0.0)
    o_ref[0] = (
        jnp.dot(h, w_ref[0], preferred_element_type=jnp.float32) + b_ref[0]
    ).astype(jnp.bfloat16)


_PERM = np.array(
    [32 * g + 16 * h + p for g in range(2) for p in range(16) for h in range(2)],
    dtype=np.int32,
)


def _split_wb(W, b):
    Ws = jnp.stack([W[:, 0:DH], W[:, DH:D]])[:, :, _PERM]
    bs = b.reshape(NC, 1, DH)[:, :, _PERM]
    return Ws, bs


def _mm1(x, W, b):
    Ws, bs = _split_wb(W, b)
    return pl.pallas_call(
        _mm1_body,
        grid=(NC,),
        in_specs=[
            pl.BlockSpec((N_NODES, D), lambda c: (0, 0)),
            pl.BlockSpec((1, D, DH), lambda c: (c, 0, 0)),
            pl.BlockSpec((1, 1, DH), lambda c: (c, 0, 0)),
        ],
        out_specs=pl.BlockSpec((1, N_NODES, DH), lambda c: (c, 0, 0)),
        out_shape=jax.ShapeDtypeStruct((NC, N_NODES, DH), jnp.bfloat16),
    )(x, Ws, bs)


def _mm2(p, W, b):
    Ws, bs = _split_wb(W, b)
    return pl.pallas_call(
        _mm2_body,
        grid=(NC,),
        in_specs=[
            pl.BlockSpec((NPAD, D), lambda c: (0, 0)),
            pl.BlockSpec((1, D, DH), lambda c: (c, 0, 0)),
            pl.BlockSpec((1, 1, DH), lambda c: (c, 0, 0)),
        ],
        out_specs=pl.BlockSpec((1, NPAD, DH), lambda c: (c, 0, 0)),
        out_shape=jax.ShapeDtypeStruct((NC, NPAD, DH), jnp.bfloat16),
    )(p, Ws, bs)


@jax.jit
def kernel(x, edge_index, edge_weight, W1, b1, W2, b2):
    dst = edge_index[0].reshape(N_EDGES // K, K)
    src = edge_index[1].reshape(N_EDGES // K, K)
    w2d = edge_weight.reshape(N_EDGES // K, K)

    hw1 = _mm1(x, W1, b1)
    p = _spmm_l1(hw1, src, dst, w2d)
    hw2 = _mm2(p, W2, b2)
    q = _spmm_l2(hw2, src, dst, w2d)
    return q[:N_NODES]
